# Initial kernel scaffold; baseline (speedup 1.0000x reference)
import jax
import jax.numpy as jnp
from jax.experimental import pallas as pl


def kernel(tg_dec, tf_base, tf_expr, scale):
    raise NotImplementedError("write your pallas kernel here")


# speedup vs baseline: 7.0397x; 7.0397x over previous
import functools
import math

import jax
import jax.numpy as jnp
from jax.experimental import pallas as pl
from jax.experimental.pallas import tpu as pltpu

_TOPK = 32


def _pick_block(g):
    return 512 if g > 512 else g


def _body(scale_ref, tg_ref, tfb_ref, tfe_ref, out_ref, attn_ref, *, d):
    sim = jax.lax.dot_general(
        tg_ref[...], tfb_ref[...], (((1,), (1,)), ((), ())),
        preferred_element_type=jnp.float32,
    ) * (1.0 / math.sqrt(d))
    m = jnp.max(sim, axis=-1, keepdims=True)
    e = jnp.exp(sim - m)
    z = jnp.sum(e, axis=-1, keepdims=True)

    def step(_, carry):
        w, _t = carry
        t = jnp.max(w, axis=-1, keepdims=True)
        w = jnp.where(w >= t, -jnp.inf, w)
        return w, t

    _, t = jax.lax.fori_loop(0, _TOPK, step, (e, jnp.zeros_like(z)))
    masked = jnp.where(e >= t, e, 0.0)
    s = jnp.sum(masked, axis=-1, keepdims=True)
    attn = masked * (1.0 / (s + 1e-8 * z))
    attn_ref[...] = attn
    out = jax.lax.dot_general(
        tfe_ref[...], attn, (((1,), (1,)), ((), ())),
        preferred_element_type=jnp.float32,
    )
    out_ref[...] = scale_ref[0, 0] * out


def kernel(tg_dec, tf_base, tf_expr, scale):
    g, d = tg_dec.shape
    t_dim = tf_base.shape[0]
    p = tf_expr.shape[0]
    r = _pick_block(g)
    scale2 = jnp.asarray(scale, jnp.float32).reshape(1, 1)
    fn = pl.pallas_call(
        functools.partial(_body, d=d),
        grid=((g + r - 1) // r,),
        in_specs=[
            pl.BlockSpec((1, 1), lambda i: (0, 0), memory_space=pltpu.SMEM),
            pl.BlockSpec((r, d), lambda i: (i, 0)),
            pl.BlockSpec((t_dim, d), lambda i: (0, 0)),
            pl.BlockSpec((p, t_dim), lambda i: (0, 0)),
        ],
        out_specs=(
            pl.BlockSpec((p, r), lambda i: (0, i)),
            pl.BlockSpec((r, t_dim), lambda i: (i, 0)),
        ),
        out_shape=(
            jax.ShapeDtypeStruct((p, g), jnp.float32),
            jax.ShapeDtypeStruct((g, t_dim), jnp.float32),
        ),
        compiler_params=pltpu.CompilerParams(
            dimension_semantics=("parallel",),
        ),
    )
    tf_scalar, attn = fn(scale2, tg_dec, tf_base, tf_expr)
    return tf_scalar, attn


# speedup vs baseline: 11.6134x; 1.6497x over previous
import functools
import math

import jax
import jax.numpy as jnp
from jax.experimental import pallas as pl
from jax.experimental.pallas import tpu as pltpu

_TOPK = 32


def _pick_block(g):
    return 512 if g > 512 else g


def _body(scale_ref, tg_ref, tfb_ref, tfe_ref, out_ref, attn_ref, *, d):
    sim = jax.lax.dot_general(
        tg_ref[...], tfb_ref[...], (((1,), (1,)), ((), ())),
        preferred_element_type=jnp.float32,
    ) * (1.0 / math.sqrt(d))
    m = jnp.max(sim, axis=-1, keepdims=True)
    e = jnp.exp(sim - m)
    z = jnp.sum(e, axis=-1, keepdims=True)

    def step(_, t):
        return jnp.max(jnp.where(e < t, e, -1.0), axis=-1, keepdims=True)

    t = jax.lax.fori_loop(0, _TOPK, step, jnp.full_like(z, jnp.inf))
    masked = jnp.where(e >= t, e, 0.0)
    s = jnp.sum(masked, axis=-1, keepdims=True)
    attn = masked * (1.0 / (s + 1e-8 * z))
    attn_ref[...] = attn
    out = jax.lax.dot_general(
        tfe_ref[...], attn, (((1,), (1,)), ((), ())),
        preferred_element_type=jnp.float32,
    )
    out_ref[...] = scale_ref[0, 0] * out


def kernel(tg_dec, tf_base, tf_expr, scale):
    g, d = tg_dec.shape
    t_dim = tf_base.shape[0]
    p = tf_expr.shape[0]
    r = _pick_block(g)
    scale2 = jnp.asarray(scale, jnp.float32).reshape(1, 1)
    fn = pl.pallas_call(
        functools.partial(_body, d=d),
        grid=((g + r - 1) // r,),
        in_specs=[
            pl.BlockSpec((1, 1), lambda i: (0, 0), memory_space=pltpu.SMEM),
            pl.BlockSpec((r, d), lambda i: (i, 0)),
            pl.BlockSpec((t_dim, d), lambda i: (0, 0)),
            pl.BlockSpec((p, t_dim), lambda i: (0, 0)),
        ],
        out_specs=(
            pl.BlockSpec((p, r), lambda i: (0, i)),
            pl.BlockSpec((r, t_dim), lambda i: (i, 0)),
        ),
        out_shape=(
            jax.ShapeDtypeStruct((p, g), jnp.float32),
            jax.ShapeDtypeStruct((g, t_dim), jnp.float32),
        ),
        compiler_params=pltpu.CompilerParams(
            dimension_semantics=("parallel",),
        ),
    )
    tf_scalar, attn = fn(scale2, tg_dec, tf_base, tf_expr)
    return tf_scalar, attn
